# Initial kernel scaffold; baseline (speedup 1.0000x reference)
import jax
import jax.numpy as jnp
from jax.experimental import pallas as pl


def kernel(inputx, adj_A, adj_A2, weight_A, weight_A2, weight_mlp, W_k0, W_k1, W_k2, att_vec_A, att_vec_A2, att_vec_mlp, att_vec):
    raise NotImplementedError("write your pallas kernel here")


# speedup vs baseline: 1.0856x; 1.0856x over previous
import functools

import jax
import jax.numpy as jnp
from jax.experimental import pallas as pl

N = 10000
D = 128

B_HEAVY = 200
B_ATT = 1000


def _stage1(x_ref, wa_ref, wa2_ref, wm_ref,
            xa_ref, xa2_ref, xm_ref, colm_ref):
    x = x_ref[...]
    xa = jnp.dot(x, wa_ref[...], preferred_element_type=jnp.float32)
    xa_ref[...] = xa.astype(jnp.bfloat16)
    xa2 = jnp.dot(x, wa2_ref[...], preferred_element_type=jnp.float32)
    xa2_ref[...] = xa2.astype(jnp.bfloat16)
    xm = jnp.maximum(jnp.dot(x, wm_ref[...], preferred_element_type=jnp.float32), 0.0)
    xm_ref[...] = xm
    colm_ref[...] = jnp.sum(xm, axis=0, keepdims=True)


def _stage2(adja_ref, adja2_ref, xa_ref, xa2_ref,
            outa_ref, outa2_ref, cola_ref, cola2_ref):
    i = pl.program_id(0)

    @pl.when(i == 0)
    def _init():
        cola_ref[...] = jnp.zeros_like(cola_ref)
        cola2_ref[...] = jnp.zeros_like(cola2_ref)

    a = adja_ref[...].astype(jnp.bfloat16)
    oa = jax.lax.dot_general(a, xa_ref[...], (((1,), (0,)), ((), ())),
                             preferred_element_type=jnp.float32)
    oa = jnp.maximum(oa, 0.0)
    outa_ref[...] = oa
    cola_ref[...] += jnp.sum(oa, axis=0, keepdims=True)

    a2 = adja2_ref[...].astype(jnp.bfloat16)
    oa2 = jax.lax.dot_general(a2, xa2_ref[...], (((1,), (0,)), ((), ())),
                              preferred_element_type=jnp.float32)
    oa2 = jnp.maximum(oa2, 0.0)
    outa2_ref[...] = oa2
    cola2_ref[...] += jnp.sum(oa2, axis=0, keepdims=True)


def _stage3(outa_ref, outa2_ref, outm_ref,
            cola_ref, cola2_ref, colm_ref,
            wk0_ref, wk1_ref, wk2_ref,
            ava_ref, ava2_ref, avm_ref, av_ref,
            out_ref):
    inv_n = 1.0 / N

    def v_vec(col_ref, wk_ref, att_ref):
        k = jnp.dot(col_ref[...] * inv_n, wk_ref[...],
                    preferred_element_type=jnp.float32)
        return jax.lax.dot_general(k, att_ref[...], (((1,), (1,)), ((), ())),
                                   preferred_element_type=jnp.float32)

    v0 = v_vec(cola_ref, wk0_ref, ava_ref)
    v1 = v_vec(cola2_ref, wk1_ref, ava2_ref)
    v2 = v_vec(colm_ref, wk2_ref, avm_ref)

    oa = outa_ref[...]
    oa2 = outa2_ref[...]
    om = outm_ref[...]

    s0 = jax.nn.sigmoid(jnp.sum(oa * v0, axis=1, keepdims=True))
    s1 = jax.nn.sigmoid(jnp.sum(oa2 * v1, axis=1, keepdims=True))
    s2 = jax.nn.sigmoid(jnp.sum(om * v2, axis=1, keepdims=True))

    av = av_ref[...]
    z = (s0 * av[0:1, :] + s1 * av[1:2, :] + s2 * av[2:3, :]) * (1.0 / 3.0)
    z = z - jnp.max(z, axis=1, keepdims=True)
    e = jnp.exp(z)
    att = e / jnp.sum(e, axis=1, keepdims=True)

    out_ref[...] = 3.0 * (att[:, 0:1] * oa + att[:, 1:2] * oa2 + att[:, 2:3] * om)


@functools.partial(jax.jit, static_argnums=())
def kernel(inputx, adj_A, adj_A2, weight_A, weight_A2, weight_mlp,
           W_k0, W_k1, W_k2, att_vec_A, att_vec_A2, att_vec_mlp, att_vec):
    f32 = jnp.float32

    xa, xa2, xm, colm = pl.pallas_call(
        _stage1,
        out_shape=(
            jax.ShapeDtypeStruct((N, D), jnp.bfloat16),
            jax.ShapeDtypeStruct((N, D), jnp.bfloat16),
            jax.ShapeDtypeStruct((N, D), f32),
            jax.ShapeDtypeStruct((1, D), f32),
        ),
    )(inputx, weight_A, weight_A2, weight_mlp)

    nblk = N // B_HEAVY
    outa, outa2, cola, cola2 = pl.pallas_call(
        _stage2,
        grid=(nblk,),
        in_specs=[
            pl.BlockSpec((B_HEAVY, N), lambda i: (i, 0)),
            pl.BlockSpec((B_HEAVY, N), lambda i: (i, 0)),
            pl.BlockSpec((N, D), lambda i: (0, 0)),
            pl.BlockSpec((N, D), lambda i: (0, 0)),
        ],
        out_specs=(
            pl.BlockSpec((B_HEAVY, D), lambda i: (i, 0)),
            pl.BlockSpec((B_HEAVY, D), lambda i: (i, 0)),
            pl.BlockSpec((1, D), lambda i: (0, 0)),
            pl.BlockSpec((1, D), lambda i: (0, 0)),
        ),
        out_shape=(
            jax.ShapeDtypeStruct((N, D), f32),
            jax.ShapeDtypeStruct((N, D), f32),
            jax.ShapeDtypeStruct((1, D), f32),
            jax.ShapeDtypeStruct((1, D), f32),
        ),
    )(adj_A, adj_A2, xa, xa2)

    natt = N // B_ATT
    out = pl.pallas_call(
        _stage3,
        grid=(natt,),
        in_specs=[
            pl.BlockSpec((B_ATT, D), lambda i: (i, 0)),
            pl.BlockSpec((B_ATT, D), lambda i: (i, 0)),
            pl.BlockSpec((B_ATT, D), lambda i: (i, 0)),
            pl.BlockSpec((1, D), lambda i: (0, 0)),
            pl.BlockSpec((1, D), lambda i: (0, 0)),
            pl.BlockSpec((1, D), lambda i: (0, 0)),
            pl.BlockSpec((D, D), lambda i: (0, 0)),
            pl.BlockSpec((D, D), lambda i: (0, 0)),
            pl.BlockSpec((D, D), lambda i: (0, 0)),
            pl.BlockSpec((D, D), lambda i: (0, 0)),
            pl.BlockSpec((D, D), lambda i: (0, 0)),
            pl.BlockSpec((D, D), lambda i: (0, 0)),
            pl.BlockSpec((3, 3), lambda i: (0, 0)),
        ],
        out_specs=pl.BlockSpec((B_ATT, D), lambda i: (i, 0)),
        out_shape=jax.ShapeDtypeStruct((N, D), f32),
    )(outa, outa2, xm, cola, cola2, colm,
      W_k0, W_k1, W_k2, att_vec_A, att_vec_A2, att_vec_mlp, att_vec)

    return out


# speedup vs baseline: 1.1293x; 1.0402x over previous
import jax
import jax.numpy as jnp
from jax.experimental import pallas as pl
from jax.experimental.pallas import tpu as pltpu

N = 10000
D = 128

B_HEAVY = 200
NBLK = N // B_HEAVY
B_ATT = 1000
NATT = N // B_ATT


def _fused(adja_ref, adja2_ref, x_ref, wa_ref, wa2_ref, wm_ref,
           wk0_ref, wk1_ref, wk2_ref, ava_ref, ava2_ref, avm_ref, av_ref,
           out_ref,
           xa_s, xa2_s, oa_s, oa2_s, xm_s, cola_s, cola2_s, colm_s):
    i = pl.program_id(0)

    @pl.when(i == 0)
    def _init():
        wa = wa_ref[...].astype(jnp.bfloat16)
        wa2 = wa2_ref[...].astype(jnp.bfloat16)

        def body(c, carry):
            rows = pl.ds(c * B_ATT, B_ATT)
            xc = x_ref[rows, :].astype(jnp.bfloat16)
            xa_s[rows, :] = jnp.dot(xc, wa,
                                    preferred_element_type=jnp.float32
                                    ).astype(jnp.bfloat16)
            xa2_s[rows, :] = jnp.dot(xc, wa2,
                                     preferred_element_type=jnp.float32
                                     ).astype(jnp.bfloat16)
            return carry

        jax.lax.fori_loop(0, NATT, body, 0)
        cola_s[...] = jnp.zeros_like(cola_s)
        cola2_s[...] = jnp.zeros_like(cola2_s)
        colm_s[...] = jnp.zeros_like(colm_s)

    @pl.when(i < NBLK)
    def _heavy():
        rows = pl.ds(i * B_HEAVY, B_HEAVY)

        xm = jnp.dot(x_ref[rows, :].astype(jnp.bfloat16),
                     wm_ref[...].astype(jnp.bfloat16),
                     preferred_element_type=jnp.float32)
        xm = jnp.maximum(xm, 0.0)
        xm_s[rows, :] = xm.astype(jnp.bfloat16)
        colm_s[...] += jnp.sum(xm, axis=0, keepdims=True)

        a = adja_ref[...].astype(jnp.bfloat16)
        oa = jax.lax.dot_general(a, xa_s[...], (((1,), (0,)), ((), ())),
                                 preferred_element_type=jnp.float32)
        oa = jnp.maximum(oa, 0.0)
        oa_s[rows, :] = oa.astype(jnp.bfloat16)
        cola_s[...] += jnp.sum(oa, axis=0, keepdims=True)

        a2 = adja2_ref[...].astype(jnp.bfloat16)
        oa2 = jax.lax.dot_general(a2, xa2_s[...], (((1,), (0,)), ((), ())),
                                  preferred_element_type=jnp.float32)
        oa2 = jnp.maximum(oa2, 0.0)
        oa2_s[rows, :] = oa2.astype(jnp.bfloat16)
        cola2_s[...] += jnp.sum(oa2, axis=0, keepdims=True)

    @pl.when(i >= NBLK)
    def _attention():
        inv_n = 1.0 / N
        rows = pl.ds((i - NBLK) * B_ATT, B_ATT)

        def v_vec(col_s, wk_ref, att_ref):
            k = jnp.dot(col_s[...] * inv_n, wk_ref[...],
                        preferred_element_type=jnp.float32)
            return jax.lax.dot_general(k, att_ref[...],
                                       (((1,), (1,)), ((), ())),
                                       preferred_element_type=jnp.float32)

        v0 = v_vec(cola_s, wk0_ref, ava_ref)
        v1 = v_vec(cola2_s, wk1_ref, ava2_ref)
        v2 = v_vec(colm_s, wk2_ref, avm_ref)

        oa = oa_s[rows, :].astype(jnp.float32)
        oa2 = oa2_s[rows, :].astype(jnp.float32)
        xm = xm_s[rows, :].astype(jnp.float32)

        s0 = jax.nn.sigmoid(jnp.sum(oa * v0, axis=1, keepdims=True))
        s1 = jax.nn.sigmoid(jnp.sum(oa2 * v1, axis=1, keepdims=True))
        s2 = jax.nn.sigmoid(jnp.sum(xm * v2, axis=1, keepdims=True))

        av = av_ref[...]
        z = (s0 * av[0:1, :] + s1 * av[1:2, :] + s2 * av[2:3, :]) * (1.0 / 3.0)
        z = z - jnp.max(z, axis=1, keepdims=True)
        e = jnp.exp(z)
        att = e / jnp.sum(e, axis=1, keepdims=True)

        out_ref[...] = 3.0 * (att[:, 0:1] * oa + att[:, 1:2] * oa2
                              + att[:, 2:3] * xm)


def kernel(inputx, adj_A, adj_A2, weight_A, weight_A2, weight_mlp,
           W_k0, W_k1, W_k2, att_vec_A, att_vec_A2, att_vec_mlp, att_vec):
    f32 = jnp.float32
    last_blk = NBLK - 1

    def adj_map(i):
        return (jnp.minimum(i, last_blk), 0)

    def out_map(i):
        return (jnp.maximum(i - NBLK, 0), 0)

    const = lambda i: (0, 0)

    out = pl.pallas_call(
        _fused,
        grid=(NBLK + NATT,),
        in_specs=[
            pl.BlockSpec((B_HEAVY, N), adj_map),
            pl.BlockSpec((B_HEAVY, N), adj_map),
            pl.BlockSpec((N, D), const),
            pl.BlockSpec((D, D), const),
            pl.BlockSpec((D, D), const),
            pl.BlockSpec((D, D), const),
            pl.BlockSpec((D, D), const),
            pl.BlockSpec((D, D), const),
            pl.BlockSpec((D, D), const),
            pl.BlockSpec((D, D), const),
            pl.BlockSpec((D, D), const),
            pl.BlockSpec((D, D), const),
            pl.BlockSpec((3, 3), const),
        ],
        out_specs=pl.BlockSpec((B_ATT, D), out_map),
        out_shape=jax.ShapeDtypeStruct((N, D), f32),
        scratch_shapes=[
            pltpu.VMEM((N, D), jnp.bfloat16),
            pltpu.VMEM((N, D), jnp.bfloat16),
            pltpu.VMEM((N, D), jnp.bfloat16),
            pltpu.VMEM((N, D), jnp.bfloat16),
            pltpu.VMEM((N, D), jnp.bfloat16),
            pltpu.VMEM((1, D), f32),
            pltpu.VMEM((1, D), f32),
            pltpu.VMEM((1, D), f32),
        ],
    )(adj_A, adj_A2, inputx, weight_A, weight_A2, weight_mlp,
      W_k0, W_k1, W_k2, att_vec_A, att_vec_A2, att_vec_mlp, att_vec)

    return out


# speedup vs baseline: 1.1395x; 1.0090x over previous
import jax
import jax.numpy as jnp
from jax.experimental import pallas as pl
from jax.experimental.pallas import tpu as pltpu

N = 10000
D = 128

B_HEAVY = 200
NBLK = N // B_HEAVY
B_ATT = 2000
NATT = N // B_ATT


def _fused(adja_ref, adja2_ref, x_ref, wa_ref, wa2_ref, wm_ref,
           wk0_ref, wk1_ref, wk2_ref, ava_ref, ava2_ref, avm_ref, av_ref,
           out_ref,
           xa_s, xa2_s, oa_s, oa2_s, xm_s, cola_s, cola2_s, colm_s, v_s):
    i = pl.program_id(0)

    @pl.when(i == 0)
    def _init():
        wa = wa_ref[...].astype(jnp.bfloat16)
        wa2 = wa2_ref[...].astype(jnp.bfloat16)

        def body(c, carry):
            rows = pl.ds(c * 1000, 1000)
            xc = x_ref[rows, :].astype(jnp.bfloat16)
            xa_s[rows, :] = jnp.dot(xc, wa,
                                    preferred_element_type=jnp.float32
                                    ).astype(jnp.bfloat16)
            xa2_s[rows, :] = jnp.dot(xc, wa2,
                                     preferred_element_type=jnp.float32
                                     ).astype(jnp.bfloat16)
            return carry

        jax.lax.fori_loop(0, N // 1000, body, 0)
        cola_s[...] = jnp.zeros_like(cola_s)
        cola2_s[...] = jnp.zeros_like(cola2_s)
        colm_s[...] = jnp.zeros_like(colm_s)

    @pl.when(i < NBLK)
    def _heavy():
        rows = pl.ds(i * B_HEAVY, B_HEAVY)

        xm = jnp.dot(x_ref[rows, :].astype(jnp.bfloat16),
                     wm_ref[...].astype(jnp.bfloat16),
                     preferred_element_type=jnp.float32)
        xm = jnp.maximum(xm, 0.0)
        xm_s[rows, :] = xm.astype(jnp.bfloat16)
        colm_s[...] += jnp.sum(xm, axis=0, keepdims=True)

        a = adja_ref[...].astype(jnp.bfloat16)
        oa = jax.lax.dot_general(a, xa_s[...], (((1,), (0,)), ((), ())),
                                 preferred_element_type=jnp.float32)
        oa = jnp.maximum(oa, 0.0)
        oa_s[rows, :] = oa.astype(jnp.bfloat16)
        cola_s[...] += jnp.sum(oa, axis=0, keepdims=True)

        a2 = adja2_ref[...].astype(jnp.bfloat16)
        oa2 = jax.lax.dot_general(a2, xa2_s[...], (((1,), (0,)), ((), ())),
                                  preferred_element_type=jnp.float32)
        oa2 = jnp.maximum(oa2, 0.0)
        oa2_s[rows, :] = oa2.astype(jnp.bfloat16)
        cola2_s[...] += jnp.sum(oa2, axis=0, keepdims=True)

    @pl.when(i == NBLK)
    def _keys():
        inv_n = 1.0 / N

        def v_vec(col_s, wk_ref, att_ref):
            k = jnp.dot(col_s[...] * inv_n, wk_ref[...],
                        preferred_element_type=jnp.float32)
            return jax.lax.dot_general(k, att_ref[...],
                                       (((1,), (1,)), ((), ())),
                                       preferred_element_type=jnp.float32)

        v_s[0:1, :] = v_vec(cola_s, wk0_ref, ava_ref)
        v_s[1:2, :] = v_vec(cola2_s, wk1_ref, ava2_ref)
        v_s[2:3, :] = v_vec(colm_s, wk2_ref, avm_ref)

    @pl.when(i >= NBLK)
    def _attention():
        rows = pl.ds((i - NBLK) * B_ATT, B_ATT)
        v0 = v_s[0:1, :]
        v1 = v_s[1:2, :]
        v2 = v_s[2:3, :]

        oa = oa_s[rows, :].astype(jnp.float32)
        oa2 = oa2_s[rows, :].astype(jnp.float32)
        xm = xm_s[rows, :].astype(jnp.float32)

        s0 = jax.nn.sigmoid(jnp.sum(oa * v0, axis=1, keepdims=True))
        s1 = jax.nn.sigmoid(jnp.sum(oa2 * v1, axis=1, keepdims=True))
        s2 = jax.nn.sigmoid(jnp.sum(xm * v2, axis=1, keepdims=True))

        av = av_ref[...]
        z = (s0 * av[0:1, :] + s1 * av[1:2, :] + s2 * av[2:3, :]) * (1.0 / 3.0)
        z = z - jnp.max(z, axis=1, keepdims=True)
        e = jnp.exp(z)
        att = e / jnp.sum(e, axis=1, keepdims=True)

        out_ref[...] = 3.0 * (att[:, 0:1] * oa + att[:, 1:2] * oa2
                              + att[:, 2:3] * xm)


def kernel(inputx, adj_A, adj_A2, weight_A, weight_A2, weight_mlp,
           W_k0, W_k1, W_k2, att_vec_A, att_vec_A2, att_vec_mlp, att_vec):
    f32 = jnp.float32
    last_blk = NBLK - 1

    def adj_map(i):
        return (jnp.minimum(i, last_blk), 0)

    def out_map(i):
        return (jnp.maximum(i - NBLK, 0), 0)

    const = lambda i: (0, 0)

    out = pl.pallas_call(
        _fused,
        grid=(NBLK + NATT,),
        in_specs=[
            pl.BlockSpec((B_HEAVY, N), adj_map),
            pl.BlockSpec((B_HEAVY, N), adj_map),
            pl.BlockSpec((N, D), const),
            pl.BlockSpec((D, D), const),
            pl.BlockSpec((D, D), const),
            pl.BlockSpec((D, D), const),
            pl.BlockSpec((D, D), const),
            pl.BlockSpec((D, D), const),
            pl.BlockSpec((D, D), const),
            pl.BlockSpec((D, D), const),
            pl.BlockSpec((D, D), const),
            pl.BlockSpec((D, D), const),
            pl.BlockSpec((3, 3), const),
        ],
        out_specs=pl.BlockSpec((B_ATT, D), out_map),
        out_shape=jax.ShapeDtypeStruct((N, D), f32),
        scratch_shapes=[
            pltpu.VMEM((N, D), jnp.bfloat16),
            pltpu.VMEM((N, D), jnp.bfloat16),
            pltpu.VMEM((N, D), jnp.bfloat16),
            pltpu.VMEM((N, D), jnp.bfloat16),
            pltpu.VMEM((N, D), jnp.bfloat16),
            pltpu.VMEM((1, D), f32),
            pltpu.VMEM((1, D), f32),
            pltpu.VMEM((1, D), f32),
            pltpu.VMEM((8, D), f32),
        ],
    )(adj_A, adj_A2, inputx, weight_A, weight_A2, weight_mlp,
      W_k0, W_k1, W_k2, att_vec_A, att_vec_A2, att_vec_mlp, att_vec)

    return out


# speedup vs baseline: 1.1425x; 1.0026x over previous
import jax
import jax.numpy as jnp
from jax.experimental import pallas as pl
from jax.experimental.pallas import tpu as pltpu

N = 10000
D = 128

B_HEAVY = 200
NBLK = N // B_HEAVY
B_ATT = 2000
NATT = N // B_ATT


def _fused(adja_ref, adja2_ref, x_ref, wa_ref, wa2_ref, wm_ref,
           wk0_ref, wk1_ref, wk2_ref, ava_ref, ava2_ref, avm_ref, av_ref,
           out_ref,
           xa_s, xa2_s, oa_s, oa2_s, cola_s, cola2_s, colm_s, v_s):
    i = pl.program_id(0)

    @pl.when(i == 0)
    def _init():
        wa = wa_ref[...]
        wa2 = wa2_ref[...]

        def body(c, carry):
            rows = pl.ds(c * 1000, 1000)
            xc = x_ref[rows, :]
            xa_s[rows, :] = jnp.dot(xc, wa, precision=jax.lax.Precision.DEFAULT,
                                    preferred_element_type=jnp.float32)
            xa2_s[rows, :] = jnp.dot(xc, wa2, precision=jax.lax.Precision.DEFAULT,
                                     preferred_element_type=jnp.float32)
            return carry

        jax.lax.fori_loop(0, N // 1000, body, 0)
        cola_s[...] = jnp.zeros_like(cola_s)
        cola2_s[...] = jnp.zeros_like(cola2_s)
        colm_s[...] = jnp.zeros_like(colm_s)

    @pl.when(i < NBLK)
    def _heavy():
        rows = pl.ds(i * B_HEAVY, B_HEAVY)

        xm = jnp.dot(x_ref[rows, :], wm_ref[...],
                     precision=jax.lax.Precision.DEFAULT,
                     preferred_element_type=jnp.float32)
        xm = jnp.maximum(xm, 0.0)
        colm_s[...] += jnp.sum(xm, axis=0, keepdims=True)

        oa = jax.lax.dot_general(adja_ref[...], xa_s[...],
                                 (((1,), (0,)), ((), ())),
                                 precision=jax.lax.Precision.DEFAULT,
                                 preferred_element_type=jnp.float32)
        oa = jnp.maximum(oa, 0.0)
        oa_s[rows, :] = oa.astype(jnp.bfloat16)
        cola_s[...] += jnp.sum(oa, axis=0, keepdims=True)

        oa2 = jax.lax.dot_general(adja2_ref[...], xa2_s[...],
                                  (((1,), (0,)), ((), ())),
                                  precision=jax.lax.Precision.DEFAULT,
                                  preferred_element_type=jnp.float32)
        oa2 = jnp.maximum(oa2, 0.0)
        oa2_s[rows, :] = oa2.astype(jnp.bfloat16)
        cola2_s[...] += jnp.sum(oa2, axis=0, keepdims=True)

    @pl.when(i == NBLK)
    def _keys():
        inv_n = 1.0 / N

        def v_vec(col_s, wk_ref, att_ref):
            k = jnp.dot(col_s[...] * inv_n, wk_ref[...],
                        preferred_element_type=jnp.float32)
            return jax.lax.dot_general(k, att_ref[...],
                                       (((1,), (1,)), ((), ())),
                                       preferred_element_type=jnp.float32)

        v_s[0:1, :] = v_vec(cola_s, wk0_ref, ava_ref)
        v_s[1:2, :] = v_vec(cola2_s, wk1_ref, ava2_ref)
        v_s[2:3, :] = v_vec(colm_s, wk2_ref, avm_ref)

    @pl.when(i >= NBLK)
    def _attention():
        rows = pl.ds((i - NBLK) * B_ATT, B_ATT)
        v0 = v_s[0:1, :]
        v1 = v_s[1:2, :]
        v2 = v_s[2:3, :]

        oa = oa_s[rows, :].astype(jnp.float32)
        oa2 = oa2_s[rows, :].astype(jnp.float32)
        xm = jnp.maximum(jnp.dot(x_ref[rows, :], wm_ref[...],
                                 precision=jax.lax.Precision.DEFAULT,
                                 preferred_element_type=jnp.float32), 0.0)

        s0 = jax.nn.sigmoid(jnp.sum(oa * v0, axis=1, keepdims=True))
        s1 = jax.nn.sigmoid(jnp.sum(oa2 * v1, axis=1, keepdims=True))
        s2 = jax.nn.sigmoid(jnp.sum(xm * v2, axis=1, keepdims=True))

        av = av_ref[...]
        z = (s0 * av[0:1, :] + s1 * av[1:2, :] + s2 * av[2:3, :]) * (1.0 / 3.0)
        z = z - jnp.max(z, axis=1, keepdims=True)
        e = jnp.exp(z)
        att = e / jnp.sum(e, axis=1, keepdims=True)

        out_ref[...] = 3.0 * (att[:, 0:1] * oa + att[:, 1:2] * oa2
                              + att[:, 2:3] * xm)


def kernel(inputx, adj_A, adj_A2, weight_A, weight_A2, weight_mlp,
           W_k0, W_k1, W_k2, att_vec_A, att_vec_A2, att_vec_mlp, att_vec):
    f32 = jnp.float32
    last_blk = NBLK - 1

    def adj_map(i):
        return (jnp.minimum(i, last_blk), 0)

    def out_map(i):
        return (jnp.maximum(i - NBLK, 0), 0)

    const = lambda i: (0, 0)

    out = pl.pallas_call(
        _fused,
        grid=(NBLK + NATT,),
        in_specs=[
            pl.BlockSpec((B_HEAVY, N), adj_map),
            pl.BlockSpec((B_HEAVY, N), adj_map),
            pl.BlockSpec((N, D), const),
            pl.BlockSpec((D, D), const),
            pl.BlockSpec((D, D), const),
            pl.BlockSpec((D, D), const),
            pl.BlockSpec((D, D), const),
            pl.BlockSpec((D, D), const),
            pl.BlockSpec((D, D), const),
            pl.BlockSpec((D, D), const),
            pl.BlockSpec((D, D), const),
            pl.BlockSpec((D, D), const),
            pl.BlockSpec((3, 3), const),
        ],
        out_specs=pl.BlockSpec((B_ATT, D), out_map),
        out_shape=jax.ShapeDtypeStruct((N, D), f32),
        scratch_shapes=[
            pltpu.VMEM((N, D), f32),
            pltpu.VMEM((N, D), f32),
            pltpu.VMEM((N, D), jnp.bfloat16),
            pltpu.VMEM((N, D), jnp.bfloat16),
            pltpu.VMEM((1, D), f32),
            pltpu.VMEM((1, D), f32),
            pltpu.VMEM((1, D), f32),
            pltpu.VMEM((8, D), f32),
        ],
    )(adj_A, adj_A2, inputx, weight_A, weight_A2, weight_mlp,
      W_k0, W_k1, W_k2, att_vec_A, att_vec_A2, att_vec_mlp, att_vec)

    return out
